# Initial kernel scaffold; baseline (speedup 1.0000x reference)
import jax
import jax.numpy as jnp
from jax.experimental import pallas as pl


def kernel(features, edge_index, edge_type, W, B, fc_w, fc_b, ln_gamma, ln_beta):
    raise NotImplementedError("write your pallas kernel here")


# speedup vs baseline: 5.3211x; 5.3211x over previous
import functools

import jax
import jax.numpy as jnp
from jax import lax
from jax.experimental import pallas as pl
from jax.experimental.pallas import tpu as pltpu
from jax.experimental.pallas import tpu_sc as plsc

N = 10000
E = 320000
D = 128
R = 4
L = 2

NC = 2
NS = 16
NW = NC * NS

RN = R * N
NP = 10240

EP = E // NW
P_CH = 2000
P_NCH = EP // P_CH

ES = E // NS
S_CH = 2000
S_NCH = ES // S_CH
GB = 128
NPAD = N + 240
ROWS_PER_TILE = NPAD // NS
OUT_ROWS = 624
OUT_REM = N - NS * OUT_ROWS


def _sc_mesh():
    return plsc.VectorSubcoreMesh(
        core_axis_name="c", subcore_axis_name="s", num_cores=NC,
        num_subcores=NS)


def _deg_body(src_hbm, dst_hbm, et_hbm, out_hbm, deg_v, sv, dv, ev):
    c = lax.axis_index("c")
    s = lax.axis_index("s")
    wid = s * NC + c

    zf = jnp.zeros((16,), jnp.float32)
    ones = jnp.full((16,), 1.0, jnp.float32)

    def zero_step(k, _):
        deg_v[pl.ds(k * 16, 16)] = zf
        return 0

    lax.fori_loop(0, 2 * R * NP // 16, zero_step, 0)

    base0 = wid * EP

    def chunk(k, _):
        base = base0 + k * P_CH
        pltpu.sync_copy(src_hbm.at[pl.ds(base, P_CH)], sv)
        pltpu.sync_copy(dst_hbm.at[pl.ds(base, P_CH)], dv)
        pltpu.sync_copy(et_hbm.at[pl.ds(base, P_CH)], ev)

        def step(j, _):
            s16 = sv[pl.ds(j * 16, 16)]
            d16 = dv[pl.ds(j * 16, 16)]
            e16 = ev[pl.ds(j * 16, 16)]
            idx_out = e16 * NP + s16
            idx_in = (R * NP + e16 * NP) + d16
            plsc.addupdate_scatter(deg_v, [idx_out], ones)
            plsc.addupdate_scatter(deg_v, [idx_in], ones)
            return 0

        lax.fori_loop(0, P_CH // 16, step, 0)
        return 0

    lax.fori_loop(0, P_NCH, chunk, 0)
    pltpu.sync_copy(deg_v, out_hbm.at[pl.ds(wid * (2 * R * NP), 2 * R * NP)])


def _deg_call(src, dst, et):
    f = functools.partial(
        pl.kernel,
        out_type=jax.ShapeDtypeStruct((NW * 2 * R * NP,), jnp.float32),
        mesh=_sc_mesh(),
        scratch_types=[
            pltpu.VMEM((2 * R * NP,), jnp.float32),
            pltpu.VMEM((P_CH,), jnp.int32),
            pltpu.VMEM((P_CH,), jnp.int32),
            pltpu.VMEM((P_CH,), jnp.int32),
        ],
        compiler_params=pltpu.CompilerParams(needs_layout_passes=False),
    )(_deg_body)
    return f(src, dst, et)


def _agg_body(hs_hbm, src_hbm, dst_hbm, et_hbm, zeros_hbm, out_hbm,
              sv, dv, ev, srcbuf, dstbuf, rows_v, acc, sem):
    c = lax.axis_index("c")
    s = lax.axis_index("s")

    def reset_bufs(rbase):
        zi = jnp.zeros((16,), jnp.int32) + rbase
        di = jnp.full((16,), N, jnp.int32)
        for t in range(GB // 16):
            srcbuf[pl.ds(t * 16, 16)] = zi
            dstbuf[pl.ds(t * 16, 16)] = di

    def fire(rbase):
        pltpu.async_copy(hs_hbm.at[srcbuf], rows_v, sem).wait()
        pltpu.sync_copy(rows_v, acc.at[dstbuf], add=True)
        reset_bufs(rbase)

    for p in range(2):
        r = c + NC * p
        rbase = r * N

        pltpu.sync_copy(zeros_hbm.at[pl.ds(s * ROWS_PER_TILE, ROWS_PER_TILE)],
                        acc.at[pl.ds(s * ROWS_PER_TILE, ROWS_PER_TILE)])
        reset_bufs(rbase)
        plsc.subcore_barrier()

        def chunk(k, cnt):
            base = s * ES + k * S_CH
            pltpu.sync_copy(src_hbm.at[pl.ds(base, S_CH)], sv)
            pltpu.sync_copy(dst_hbm.at[pl.ds(base, S_CH)], dv)
            pltpu.sync_copy(et_hbm.at[pl.ds(base, S_CH)], ev)

            def step(j, cnt):
                s16 = sv[pl.ds(j * 16, 16)]
                d16 = dv[pl.ds(j * 16, 16)]
                e16 = ev[pl.ds(j * 16, 16)]
                m = e16 == r
                plsc.store_compressed(srcbuf.at[pl.ds(cnt, 16)],
                                      s16 + rbase, mask=m)
                plsc.store_compressed(dstbuf.at[pl.ds(cnt, 16)], d16, mask=m)
                cnt = cnt + jnp.sum(m.astype(jnp.int32))
                full = cnt > GB - 16

                @pl.when(full)
                def _():
                    fire(rbase)

                return jnp.where(full, 0, cnt)

            return lax.fori_loop(0, S_CH // 16, step, cnt)

        cnt = lax.fori_loop(0, S_NCH, chunk, jnp.int32(0))

        @pl.when(cnt > 0)
        def _():
            fire(rbase)

        plsc.subcore_barrier()
        pltpu.sync_copy(acc.at[pl.ds(s * OUT_ROWS, OUT_ROWS)],
                        out_hbm.at[pl.ds(rbase + s * OUT_ROWS, OUT_ROWS)])

        @pl.when(s == 0)
        def _():
            pltpu.sync_copy(
                acc.at[pl.ds(NS * OUT_ROWS, OUT_REM)],
                out_hbm.at[pl.ds(rbase + NS * OUT_ROWS, OUT_REM)])

        plsc.subcore_barrier()


def _agg_call(hs, src, dst, et, zeros_pad):
    f = functools.partial(
        pl.kernel,
        out_type=jax.ShapeDtypeStruct((RN, D), jnp.float32),
        mesh=_sc_mesh(),
        scratch_types=[
            pltpu.VMEM((S_CH,), jnp.int32),
            pltpu.VMEM((S_CH,), jnp.int32),
            pltpu.VMEM((S_CH,), jnp.int32),
            pltpu.VMEM((GB,), jnp.int32),
            pltpu.VMEM((GB,), jnp.int32),
            pltpu.VMEM((GB, D), jnp.float32),
            pltpu.VMEM_SHARED((NPAD, D), jnp.float32),
            pltpu.SemaphoreType.DMA,
        ],
        compiler_params=pltpu.CompilerParams(needs_layout_passes=False),
    )(_agg_body)
    return f(hs, src, dst, et, zeros_pad)


def _norm_body(degparts_ref, norm_ref):
    deg = jnp.sum(degparts_ref[...], axis=0)
    norm_ref[...] = lax.rsqrt(jnp.maximum(deg, 1.0))


def _norm_call(degparts):
    return pl.pallas_call(
        _norm_body,
        out_shape=jax.ShapeDtypeStruct((2 * R * NP,), jnp.float32),
    )(degparts)


NB2 = 2000


def _hs_body(feats_ref, ns_ref, hs_ref):
    feats = feats_ref[...]
    ns = ns_ref[...]
    for r in range(R):
        hs_ref[r] = feats * ns[r]


def _hs_call(features, ns_view):
    return pl.pallas_call(
        _hs_body,
        grid=(N // NB2,),
        in_specs=[
            pl.BlockSpec((NB2, D), lambda i: (i, 0)),
            pl.BlockSpec((R, NB2, 1), lambda i: (0, i, 0)),
        ],
        out_specs=pl.BlockSpec((R, NB2, D), lambda i: (0, i, 0)),
        out_shape=jax.ShapeDtypeStruct((R, N, D), jnp.float32),
    )(features, ns_view)


NB = 2000


def _layer_body(is_last, h_ref, agg_ref, nd_ref, ns_ref, w_ref, b_ref,
                fcw_ref, fcb_ref, g_ref, beta_ref, feats_ref, hout_ref,
                *maybe_hs_out):
    agg = agg_ref[...]
    nd = nd_ref[...]
    a = agg * nd
    w = w_ref[...]
    conv = jnp.zeros((NB, D), jnp.float32)
    for r in range(R):
        conv = conv + jnp.dot(a[r], w[r],
                              precision=lax.Precision.HIGHEST)
    conv = conv + jnp.sum(b_ref[...], axis=0)[None, :]
    logits = jnp.dot(conv, fcw_ref[...],
                     precision=lax.Precision.HIGHEST) + fcb_ref[0]
    attn = jax.nn.sigmoid(logits)
    h2 = h_ref[...] + attn * conv
    mu = jnp.mean(h2, axis=-1, keepdims=True)
    var = jnp.mean((h2 - mu) ** 2, axis=-1, keepdims=True)
    y = (h2 - mu) * lax.rsqrt(var + 1e-5) * g_ref[...][None, :] \
        + beta_ref[...][None, :]
    h3 = jnp.maximum(y, 0.0)
    if is_last:
        h3 = h3 + feats_ref[...]
        hout_ref[...] = h3
    else:
        hout_ref[...] = h3
        maybe_hs_out[0][...] = h3[None, :, :] * ns_ref[...]


def _layer_call(h, agg, nd, ns, w, b, fcw, fcb, g, beta, feats, is_last):
    grid = (N // NB,)
    in_specs = [
        pl.BlockSpec((NB, D), lambda i: (i, 0)),
        pl.BlockSpec((R, NB, D), lambda i: (0, i, 0)),
        pl.BlockSpec((R, NB, 1), lambda i: (0, i, 0)),
        pl.BlockSpec((R, NB, 1), lambda i: (0, i, 0)),
        pl.BlockSpec((R, D, D), lambda i: (0, 0, 0)),
        pl.BlockSpec((R, D), lambda i: (0, 0)),
        pl.BlockSpec((D, 1), lambda i: (0, 0)),
        pl.BlockSpec((1,), lambda i: (0,)),
        pl.BlockSpec((D,), lambda i: (0,)),
        pl.BlockSpec((D,), lambda i: (0,)),
        pl.BlockSpec((NB, D), lambda i: (i, 0)),
    ]
    out_shape = [jax.ShapeDtypeStruct((N, D), jnp.float32)]
    out_specs = [pl.BlockSpec((NB, D), lambda i: (i, 0))]
    if not is_last:
        out_shape.append(jax.ShapeDtypeStruct((R, N, D), jnp.float32))
        out_specs.append(pl.BlockSpec((R, NB, D), lambda i: (0, i, 0)))
    return pl.pallas_call(
        functools.partial(_layer_body, is_last),
        grid=grid,
        in_specs=in_specs,
        out_specs=out_specs,
        out_shape=out_shape,
    )(h, agg, nd, ns, w, b, fcw, fcb, g, beta, feats)


def kernel(features, edge_index, edge_type, W, B, fc_w, fc_b, ln_gamma,
           ln_beta):
    src = edge_index[0]
    dst = edge_index[1]
    et = edge_type

    degparts = _deg_call(src, dst, et).reshape(NW, 2 * R * NP)
    norm_flat = _norm_call(degparts)
    ns = norm_flat[:R * NP].reshape(R, NP, 1)
    nd = norm_flat[R * NP:].reshape(R, NP, 1)
    hs = _hs_call(features, ns).reshape(RN, D)

    zeros_pad = jnp.zeros((NPAD, D), jnp.float32)
    h = features
    for l in range(L):
        agg3 = _agg_call(hs, src, dst, et, zeros_pad).reshape(R, N, D)
        is_last = l == L - 1
        outs = _layer_call(h, agg3, nd, ns, W[l], B[l], fc_w, fc_b,
                           ln_gamma[l], ln_beta[l], features, is_last)
        if is_last:
            h = outs[0]
        else:
            h, hs4 = outs
            hs = hs4.reshape(RN, D)
    return h


# speedup vs baseline: 14.0316x; 2.6370x over previous
import functools

import jax
import jax.numpy as jnp
from jax import lax
from jax.experimental import pallas as pl
from jax.experimental.pallas import tpu as pltpu
from jax.experimental.pallas import tpu_sc as plsc

N = 10000
E = 320000
D = 128
R = 4
L = 2

NC = 2
NS = 16
NW = NC * NS

RN = R * N
NP = 10240

EP = E // NW
P_CH = 2000
P_NCH = EP // P_CH

ES = E // NS
S_CH = 4000
S_NCH = ES // S_CH
GB = 128
NPAD = N + 240
ROWS_PER_TILE = NPAD // NS
OUT_ROWS = 624
OUT_REM = N - NS * OUT_ROWS


def _sc_mesh():
    return plsc.VectorSubcoreMesh(
        core_axis_name="c", subcore_axis_name="s", num_cores=NC,
        num_subcores=NS)


LCAP = 10240


def _deg_body(pk_hbm, out_hbm, lists_hbm, cnt_hbm, deg_v, pv, lx, cv):
    c = lax.axis_index("c")
    s = lax.axis_index("s")
    wid = s * NC + c

    zf = jnp.zeros((16,), jnp.float32)
    ones = jnp.full((16,), 1.0, jnp.float32)

    def zero_step(k, _):
        deg_v[pl.ds(k * 16, 16)] = zf
        return 0

    lax.fori_loop(0, 2 * R * NP // 16, zero_step, 0)

    base0 = wid * EP

    def chunk(k, cnts):
        base = base0 + k * P_CH
        pltpu.sync_copy(pk_hbm.at[pl.ds(base, P_CH)], pv)

        def step(j, cnts):
            w16 = pv[pl.ds(j * 16, 16)]
            e16 = w16 >> 28
            s16 = (w16 >> 14) & 16383
            d16 = w16 & 16383
            idx_out = e16 * NP + s16
            idx_in = (R * NP + e16 * NP) + d16
            plsc.addupdate_scatter(deg_v, [idx_out], ones)
            plsc.addupdate_scatter(deg_v, [idx_in], ones)
            out = []
            for r in range(R):
                m = e16 == r
                plsc.store_compressed(lx.at[pl.ds(r * LCAP + cnts[r], 16)],
                                      w16, mask=m)
                out.append(cnts[r] + plsc.all_reduce_population_count(m)[0])
            return tuple(out)

        return lax.fori_loop(0, P_CH // 16, step, cnts)

    z = jnp.int32(0)
    cnts = lax.fori_loop(0, P_NCH, chunk, (z, z, z, z))

    dummy = jnp.full((16,), N, jnp.int32)
    lanes = lax.iota(jnp.int32, 16)
    cvec = jnp.zeros((16,), jnp.int32)
    for r in range(R):
        for j in range(GB // 16):
            lx[pl.ds(r * LCAP + cnts[r] + j * 16, 16)] = dummy
        cvec = jnp.where(lanes == r, cnts[r], cvec)
    cv[pl.ds(0, 16)] = cvec
    pltpu.sync_copy(cv, cnt_hbm.at[pl.ds(wid * 16, 16)])
    pltpu.sync_copy(lx, lists_hbm.at[pl.ds(wid * R * LCAP, R * LCAP)])
    pltpu.sync_copy(deg_v, out_hbm.at[pl.ds(wid * (2 * R * NP), 2 * R * NP)])


def _deg_call(packed):
    f = functools.partial(
        pl.kernel,
        out_type=[
            jax.ShapeDtypeStruct((NW * 2 * R * NP,), jnp.float32),
            jax.ShapeDtypeStruct((NW * R * LCAP,), jnp.int32),
            jax.ShapeDtypeStruct((NW * 16,), jnp.int32),
        ],
        mesh=_sc_mesh(),
        scratch_types=[
            pltpu.VMEM((2 * R * NP,), jnp.float32),
            pltpu.VMEM((P_CH,), jnp.int32),
            pltpu.VMEM((R * LCAP,), jnp.int32),
            pltpu.VMEM((16,), jnp.int32),
        ],
        compiler_params=pltpu.CompilerParams(needs_layout_passes=False),
    )(_deg_body)
    return f(packed)


NBUF = 2
S_CAP = S_CH + GB


def _agg_body(hs_hbm, lists_hbm, cnts_hbm, zeros_hbm, out_hbm,
              pkx, cv, src2, dst2, rows, acc, gsem):
    c = lax.axis_index("c")
    s = lax.axis_index("s")

    for p in range(2):
        r = c + NC * p
        rbase = r * N

        pltpu.sync_copy(zeros_hbm.at[pl.ds(s * ROWS_PER_TILE, ROWS_PER_TILE)],
                        acc.at[pl.ds(s * ROWS_PER_TILE, ROWS_PER_TILE)])
        plsc.subcore_barrier()

        def wait_gather(buf):
            pltpu.make_async_copy(hs_hbm.at[src2.at[buf]],
                                  rows.at[buf], gsem).wait()

        def seg_loop(seg, fc):
            wid = 2 * s + seg
            pltpu.sync_copy(cnts_hbm.at[pl.ds(wid * 16, 16)], cv)
            pltpu.sync_copy(
                lists_hbm.at[pl.ds((wid * R + r) * LCAP, LCAP)], pkx)
            lanes = lax.iota(jnp.int32, 16)
            cvv = cv[pl.ds(0, 16)]
            cnt = jnp.sum(jnp.where(lanes == r, cvv, 0))
            nbk = (cnt + (GB - 1)) // GB

            def fire(q, fc):
                b = lax.rem(fc, NBUF)
                for j in range(GB // 16):
                    w = pkx[pl.ds(q * GB + j * 16, 16)]
                    src2[b, pl.ds(j * 16, 16)] = ((w >> 14) & 16383) + rbase
                    dst2[b, pl.ds(j * 16, 16)] = w & 16383
                pltpu.async_copy(hs_hbm.at[src2.at[b]], rows.at[b], gsem)

                @pl.when(fc >= 1)
                def _():
                    bp = lax.rem(fc + (NBUF - 1), NBUF)
                    wait_gather(bp)
                    pltpu.sync_copy(rows.at[bp], acc.at[dst2.at[bp]],
                                    add=True)

                return fc + 1

            return lax.fori_loop(0, nbk, fire, fc)

        fc = lax.fori_loop(0, 2, seg_loop, jnp.int32(0))

        @pl.when(fc >= 1)
        def _():
            blast = lax.rem(fc + (NBUF - 1), NBUF)
            wait_gather(blast)
            pltpu.sync_copy(rows.at[blast], acc.at[dst2.at[blast]], add=True)

        plsc.subcore_barrier()
        pltpu.sync_copy(acc.at[pl.ds(s * OUT_ROWS, OUT_ROWS)],
                        out_hbm.at[pl.ds(rbase + s * OUT_ROWS, OUT_ROWS)])

        @pl.when(s == 0)
        def _():
            pltpu.sync_copy(
                acc.at[pl.ds(NS * OUT_ROWS, OUT_REM)],
                out_hbm.at[pl.ds(rbase + NS * OUT_ROWS, OUT_REM)])

        plsc.subcore_barrier()


def _agg_call(hs, lists, cnts, zeros_pad):
    f = functools.partial(
        pl.kernel,
        out_type=jax.ShapeDtypeStruct((RN, D), jnp.float32),
        mesh=_sc_mesh(),
        scratch_types=[
            pltpu.VMEM((LCAP,), jnp.int32),
            pltpu.VMEM((16,), jnp.int32),
            pltpu.VMEM((NBUF, GB), jnp.int32),
            pltpu.VMEM((NBUF, GB), jnp.int32),
            pltpu.VMEM((NBUF, GB, D), jnp.float32),
            pltpu.VMEM_SHARED((NPAD, D), jnp.float32),
            pltpu.SemaphoreType.DMA,
        ],
        compiler_params=pltpu.CompilerParams(needs_layout_passes=False),
    )(_agg_body)
    return f(hs, lists, cnts, zeros_pad)


def _norm_body(degparts_ref, norm_ref):
    seg = 2 * R * NP
    deg = degparts_ref[pl.ds(0, seg)]
    for w in range(1, NW):
        deg = deg + degparts_ref[pl.ds(w * seg, seg)]
    norm_ref[...] = lax.rsqrt(jnp.maximum(deg, 1.0))


def _norm_call(degparts):
    return pl.pallas_call(
        _norm_body,
        out_shape=jax.ShapeDtypeStruct((2 * R * NP,), jnp.float32),
    )(degparts)


NB2 = 2000


def _hs_body(feats_ref, ns_ref, hs_ref):
    feats = feats_ref[...]
    ns = ns_ref[...]
    for r in range(R):
        hs_ref[r] = feats * ns[r]


def _hs_call(features, ns_view):
    return pl.pallas_call(
        _hs_body,
        grid=(N // NB2,),
        in_specs=[
            pl.BlockSpec((NB2, D), lambda i: (i, 0)),
            pl.BlockSpec((R, NB2, 1), lambda i: (0, i, 0)),
        ],
        out_specs=pl.BlockSpec((R, NB2, D), lambda i: (0, i, 0)),
        out_shape=jax.ShapeDtypeStruct((R, N, D), jnp.float32),
    )(features, ns_view)


NB = 2000


def _layer_body(is_last, h_ref, agg_ref, nd_ref, ns_ref, w_ref, b_ref,
                fcw_ref, fcb_ref, g_ref, beta_ref, feats_ref, hout_ref,
                *maybe_hs_out):
    agg = agg_ref[...]
    nd = nd_ref[...]
    a = agg * nd
    w = w_ref[...]
    conv = jnp.zeros((NB, D), jnp.float32)
    for r in range(R):
        conv = conv + jnp.dot(a[r], w[r],
                              precision=lax.Precision.DEFAULT)
    conv = conv + jnp.sum(b_ref[...], axis=0)[None, :]
    logits = jnp.dot(conv, fcw_ref[...],
                     precision=lax.Precision.HIGHEST) + fcb_ref[0]
    attn = jax.nn.sigmoid(logits)
    h2 = h_ref[...] + attn * conv
    mu = jnp.mean(h2, axis=-1, keepdims=True)
    var = jnp.mean((h2 - mu) ** 2, axis=-1, keepdims=True)
    y = (h2 - mu) * lax.rsqrt(var + 1e-5) * g_ref[...][None, :] \
        + beta_ref[...][None, :]
    h3 = jnp.maximum(y, 0.0)
    if is_last:
        h3 = h3 + feats_ref[...]
        hout_ref[...] = h3
    else:
        hout_ref[...] = h3
        maybe_hs_out[0][...] = h3[None, :, :] * ns_ref[...]


def _layer_call(h, agg, nd, ns, w, b, fcw, fcb, g, beta, feats, is_last):
    grid = (N // NB,)
    in_specs = [
        pl.BlockSpec((NB, D), lambda i: (i, 0)),
        pl.BlockSpec((R, NB, D), lambda i: (0, i, 0)),
        pl.BlockSpec((R, NB, 1), lambda i: (0, i, 0)),
        pl.BlockSpec((R, NB, 1), lambda i: (0, i, 0)),
        pl.BlockSpec((R, D, D), lambda i: (0, 0, 0)),
        pl.BlockSpec((R, D), lambda i: (0, 0)),
        pl.BlockSpec((D, 1), lambda i: (0, 0)),
        pl.BlockSpec((1,), lambda i: (0,)),
        pl.BlockSpec((D,), lambda i: (0,)),
        pl.BlockSpec((D,), lambda i: (0,)),
        pl.BlockSpec((NB, D), lambda i: (i, 0)),
    ]
    out_shape = [jax.ShapeDtypeStruct((N, D), jnp.float32)]
    out_specs = [pl.BlockSpec((NB, D), lambda i: (i, 0))]
    if not is_last:
        out_shape.append(jax.ShapeDtypeStruct((R, N, D), jnp.float32))
        out_specs.append(pl.BlockSpec((R, NB, D), lambda i: (0, i, 0)))
    return pl.pallas_call(
        functools.partial(_layer_body, is_last),
        grid=grid,
        in_specs=in_specs,
        out_specs=out_specs,
        out_shape=out_shape,
    )(h, agg, nd, ns, w, b, fcw, fcb, g, beta, feats)


def kernel(features, edge_index, edge_type, W, B, fc_w, fc_b, ln_gamma,
           ln_beta):
    src = edge_index[0]
    dst = edge_index[1]
    packed = (edge_type.astype(jnp.int32) << 28) | (src << 14) | dst

    degflat, lists, cnts = _deg_call(packed)
    norm_flat = _norm_call(degflat)
    ns = norm_flat[:R * NP].reshape(R, NP, 1)
    nd = norm_flat[R * NP:].reshape(R, NP, 1)
    hs = _hs_call(features, ns).reshape(RN, D)

    zeros_pad = jnp.zeros((NPAD, D), jnp.float32)
    h = features
    for l in range(L):
        agg3 = _agg_call(hs, lists, cnts, zeros_pad).reshape(R, N, D)
        is_last = l == L - 1
        outs = _layer_call(h, agg3, nd, ns, W[l], B[l], fc_w, fc_b,
                           ln_gamma[l], ln_beta[l], features, is_last)
        if is_last:
            h = outs[0]
        else:
            h, hs4 = outs
            hs = hs4.reshape(RN, D)
    return h


# speedup vs baseline: 14.4036x; 1.0265x over previous
import functools

import jax
import jax.numpy as jnp
from jax import lax
from jax.experimental import pallas as pl
from jax.experimental.pallas import tpu as pltpu
from jax.experimental.pallas import tpu_sc as plsc

N = 10000
E = 320000
D = 128
R = 4
L = 2

NC = 2
NS = 16
NW = NC * NS

RN = R * N
NP = 10240

EP = E // NW
P_CH = 2000
P_NCH = EP // P_CH

ES = E // NS
S_CH = 4000
S_NCH = ES // S_CH
GB = 112
NPAD = N + 112
ROWS_PER_TILE = NPAD // NS
OUT_ROWS = 624
OUT_REM = N - NS * OUT_ROWS


def _sc_mesh():
    return plsc.VectorSubcoreMesh(
        core_axis_name="c", subcore_axis_name="s", num_cores=NC,
        num_subcores=NS)


LCAP = 11200
STG = 2240
BPS = STG // GB
NSTG = LCAP // STG


def _deg_body(pk_hbm, out_hbm, lists_hbm, cnt_hbm, deg_v, pv, lx, cv):
    c = lax.axis_index("c")
    s = lax.axis_index("s")
    wid = s * NC + c

    zf = jnp.zeros((16,), jnp.float32)
    ones = jnp.full((16,), 1.0, jnp.float32)

    def zero_step(k, _):
        deg_v[pl.ds(k * 16, 16)] = zf
        return 0

    lax.fori_loop(0, 2 * R * NP // 16, zero_step, 0)

    base0 = wid * EP

    def chunk(k, cnts):
        base = base0 + k * P_CH
        pltpu.sync_copy(pk_hbm.at[pl.ds(base, P_CH)], pv)

        def step(j, cnts):
            w16 = pv[pl.ds(j * 16, 16)]
            e16 = w16 >> 28
            s16 = (w16 >> 14) & 16383
            d16 = w16 & 16383
            idx_out = e16 * NP + s16
            idx_in = (R * NP + e16 * NP) + d16
            plsc.addupdate_scatter(deg_v, [idx_out], ones)
            plsc.addupdate_scatter(deg_v, [idx_in], ones)
            out = []
            for r in range(R):
                m = e16 == r
                plsc.store_compressed(lx.at[pl.ds(r * LCAP + cnts[r], 16)],
                                      w16, mask=m)
                out.append(cnts[r] + plsc.all_reduce_population_count(m)[0])
            return tuple(out)

        return lax.fori_loop(0, P_CH // 16, step, cnts)

    z = jnp.int32(0)
    cnts = lax.fori_loop(0, P_NCH, chunk, (z, z, z, z))

    dummy = jnp.full((16,), N, jnp.int32)
    lanes = lax.iota(jnp.int32, 16)
    cvec = jnp.zeros((16,), jnp.int32)
    for r in range(R):
        for j in range(GB // 16):
            lx[pl.ds(r * LCAP + cnts[r] + j * 16, 16)] = dummy
        cvec = jnp.where(lanes == r, cnts[r], cvec)
    cv[pl.ds(0, 16)] = cvec
    pltpu.sync_copy(cv, cnt_hbm.at[pl.ds(wid * 16, 16)])
    pltpu.sync_copy(lx, lists_hbm.at[pl.ds(wid * R * LCAP, R * LCAP)])
    pltpu.sync_copy(deg_v, out_hbm.at[pl.ds(wid * (2 * R * NP), 2 * R * NP)])


def _deg_call(packed):
    f = functools.partial(
        pl.kernel,
        out_type=[
            jax.ShapeDtypeStruct((NW * 2 * R * NP,), jnp.float32),
            jax.ShapeDtypeStruct((NW * R * LCAP,), jnp.int32),
            jax.ShapeDtypeStruct((NW * 16,), jnp.int32),
        ],
        mesh=_sc_mesh(),
        scratch_types=[
            pltpu.VMEM((2 * R * NP,), jnp.float32),
            pltpu.VMEM((P_CH,), jnp.int32),
            pltpu.VMEM((R * LCAP,), jnp.int32),
            pltpu.VMEM((16,), jnp.int32),
        ],
        compiler_params=pltpu.CompilerParams(needs_layout_passes=False),
    )(_deg_body)
    return f(packed)


NBUF = 3


def _agg_body(hs_hbm, lists_hbm, cnts_hbm, zeros_hbm, out_hbm,
              pkx, cv, src2, dst2, rows, acc, gsem):
    c = lax.axis_index("c")
    s = lax.axis_index("s")

    for p in range(2):
        r = c + NC * p
        rbase = r * N

        pltpu.sync_copy(zeros_hbm.at[pl.ds(s * ROWS_PER_TILE, ROWS_PER_TILE)],
                        acc.at[pl.ds(s * ROWS_PER_TILE, ROWS_PER_TILE)])
        plsc.subcore_barrier()

        def wait_gather(buf):
            pltpu.make_async_copy(hs_hbm.at[src2.at[buf]],
                                  rows.at[buf], gsem).wait()

        def seg_loop(seg, fc):
            wid = 2 * s + seg
            pltpu.sync_copy(cnts_hbm.at[pl.ds(wid * 16, 16)], cv)
            lanes = lax.iota(jnp.int32, 16)
            cvv = cv[pl.ds(0, 16)]
            cnt = jnp.sum(jnp.where(lanes == r, cvv, 0))
            nbt = (cnt + (GB - 1)) // GB
            lbase = (wid * R + r) * LCAP

            def stage_loop(h, fc):
                pltpu.sync_copy(
                    lists_hbm.at[pl.ds(lbase + h * STG, STG)], pkx)
                nbh = jnp.clip(nbt - h * BPS, 0, BPS)

                def fire(q, fc):
                    b = lax.rem(fc, NBUF)
                    for j in range(GB // 16):
                        w = pkx[pl.ds(q * GB + j * 16, 16)]
                        src2[b, pl.ds(j * 16, 16)] = \
                            ((w >> 14) & 16383) + rbase
                        dst2[b, pl.ds(j * 16, 16)] = w & 16383
                    pltpu.async_copy(hs_hbm.at[src2.at[b]], rows.at[b], gsem)

                    @pl.when(fc >= 1)
                    def _():
                        bp = lax.rem(fc + (NBUF - 1), NBUF)
                        wait_gather(bp)
                        pltpu.sync_copy(rows.at[bp], acc.at[dst2.at[bp]],
                                        add=True)

                    return fc + 1

                return lax.fori_loop(0, nbh, fire, fc)

            return lax.fori_loop(0, NSTG, stage_loop, fc)

        fc = lax.fori_loop(0, 2, seg_loop, jnp.int32(0))

        @pl.when(fc >= 1)
        def _():
            blast = lax.rem(fc + (NBUF - 1), NBUF)
            wait_gather(blast)
            pltpu.sync_copy(rows.at[blast], acc.at[dst2.at[blast]], add=True)

        plsc.subcore_barrier()
        pltpu.sync_copy(acc.at[pl.ds(s * OUT_ROWS, OUT_ROWS)],
                        out_hbm.at[pl.ds(rbase + s * OUT_ROWS, OUT_ROWS)])

        @pl.when(s == 0)
        def _():
            pltpu.sync_copy(
                acc.at[pl.ds(NS * OUT_ROWS, OUT_REM)],
                out_hbm.at[pl.ds(rbase + NS * OUT_ROWS, OUT_REM)])

        plsc.subcore_barrier()


def _agg_call(hs, lists, cnts, zeros_pad):
    f = functools.partial(
        pl.kernel,
        out_type=jax.ShapeDtypeStruct((RN, D), jnp.float32),
        mesh=_sc_mesh(),
        scratch_types=[
            pltpu.VMEM((STG,), jnp.int32),
            pltpu.VMEM((16,), jnp.int32),
            pltpu.VMEM((NBUF, GB), jnp.int32),
            pltpu.VMEM((NBUF, GB), jnp.int32),
            pltpu.VMEM((NBUF, GB, D), jnp.float32),
            pltpu.VMEM_SHARED((NPAD, D), jnp.float32),
            pltpu.SemaphoreType.DMA,
        ],
        compiler_params=pltpu.CompilerParams(needs_layout_passes=False),
    )(_agg_body)
    return f(hs, lists, cnts, zeros_pad)


def _norm_body(degparts_ref, norm_ref):
    seg = 2 * R * NP
    deg = degparts_ref[pl.ds(0, seg)]
    for w in range(1, NW):
        deg = deg + degparts_ref[pl.ds(w * seg, seg)]
    norm_ref[...] = lax.rsqrt(jnp.maximum(deg, 1.0))


def _norm_call(degparts):
    return pl.pallas_call(
        _norm_body,
        out_shape=jax.ShapeDtypeStruct((2 * R * NP,), jnp.float32),
    )(degparts)


NB2 = 2000


def _hs_body(feats_ref, ns_ref, hs_ref):
    feats = feats_ref[...]
    ns = ns_ref[...]
    for r in range(R):
        hs_ref[r] = feats * ns[r]


def _hs_call(features, ns_view):
    return pl.pallas_call(
        _hs_body,
        grid=(N // NB2,),
        in_specs=[
            pl.BlockSpec((NB2, D), lambda i: (i, 0)),
            pl.BlockSpec((R, NB2, 1), lambda i: (0, i, 0)),
        ],
        out_specs=pl.BlockSpec((R, NB2, D), lambda i: (0, i, 0)),
        out_shape=jax.ShapeDtypeStruct((R, N, D), jnp.float32),
    )(features, ns_view)


NB = 2000


def _layer_body(is_last, h_ref, agg_ref, nd_ref, ns_ref, w_ref, b_ref,
                fcw_ref, fcb_ref, g_ref, beta_ref, feats_ref, hout_ref,
                *maybe_hs_out):
    agg = agg_ref[...]
    nd = nd_ref[...]
    a = agg * nd
    w = w_ref[...]
    conv = jnp.zeros((NB, D), jnp.float32)
    for r in range(R):
        conv = conv + jnp.dot(a[r], w[r],
                              precision=lax.Precision.DEFAULT)
    conv = conv + jnp.sum(b_ref[...], axis=0)[None, :]
    logits = jnp.dot(conv, fcw_ref[...],
                     precision=lax.Precision.HIGHEST) + fcb_ref[0]
    attn = jax.nn.sigmoid(logits)
    h2 = h_ref[...] + attn * conv
    mu = jnp.mean(h2, axis=-1, keepdims=True)
    var = jnp.mean((h2 - mu) ** 2, axis=-1, keepdims=True)
    y = (h2 - mu) * lax.rsqrt(var + 1e-5) * g_ref[...][None, :] \
        + beta_ref[...][None, :]
    h3 = jnp.maximum(y, 0.0)
    if is_last:
        h3 = h3 + feats_ref[...]
        hout_ref[...] = h3
    else:
        hout_ref[...] = h3
        maybe_hs_out[0][...] = h3[None, :, :] * ns_ref[...]


def _layer_call(h, agg, nd, ns, w, b, fcw, fcb, g, beta, feats, is_last):
    grid = (N // NB,)
    in_specs = [
        pl.BlockSpec((NB, D), lambda i: (i, 0)),
        pl.BlockSpec((R, NB, D), lambda i: (0, i, 0)),
        pl.BlockSpec((R, NB, 1), lambda i: (0, i, 0)),
        pl.BlockSpec((R, NB, 1), lambda i: (0, i, 0)),
        pl.BlockSpec((R, D, D), lambda i: (0, 0, 0)),
        pl.BlockSpec((R, D), lambda i: (0, 0)),
        pl.BlockSpec((D, 1), lambda i: (0, 0)),
        pl.BlockSpec((1,), lambda i: (0,)),
        pl.BlockSpec((D,), lambda i: (0,)),
        pl.BlockSpec((D,), lambda i: (0,)),
        pl.BlockSpec((NB, D), lambda i: (i, 0)),
    ]
    out_shape = [jax.ShapeDtypeStruct((N, D), jnp.float32)]
    out_specs = [pl.BlockSpec((NB, D), lambda i: (i, 0))]
    if not is_last:
        out_shape.append(jax.ShapeDtypeStruct((R, N, D), jnp.float32))
        out_specs.append(pl.BlockSpec((R, NB, D), lambda i: (0, i, 0)))
    return pl.pallas_call(
        functools.partial(_layer_body, is_last),
        grid=grid,
        in_specs=in_specs,
        out_specs=out_specs,
        out_shape=out_shape,
    )(h, agg, nd, ns, w, b, fcw, fcb, g, beta, feats)


def kernel(features, edge_index, edge_type, W, B, fc_w, fc_b, ln_gamma,
           ln_beta):
    src = edge_index[0]
    dst = edge_index[1]
    packed = (edge_type.astype(jnp.int32) << 28) | (src << 14) | dst

    degflat, lists, cnts = _deg_call(packed)
    norm_flat = _norm_call(degflat)
    ns = norm_flat[:R * NP].reshape(R, NP, 1)
    nd = norm_flat[R * NP:].reshape(R, NP, 1)
    hs = _hs_call(features, ns).reshape(RN, D)

    zeros_pad = jnp.zeros((NPAD, D), jnp.float32)
    h = features
    for l in range(L):
        agg3 = _agg_call(hs, lists, cnts, zeros_pad).reshape(R, N, D)
        is_last = l == L - 1
        outs = _layer_call(h, agg3, nd, ns, W[l], B[l], fc_w, fc_b,
                           ln_gamma[l], ln_beta[l], features, is_last)
        if is_last:
            h = outs[0]
        else:
            h, hs4 = outs
            hs = hs4.reshape(RN, D)
    return h


# speedup vs baseline: 15.2830x; 1.0611x over previous
import functools

import jax
import jax.numpy as jnp
from jax import lax
from jax.experimental import pallas as pl
from jax.experimental.pallas import tpu as pltpu
from jax.experimental.pallas import tpu_sc as plsc

N = 10000
E = 320000
D = 128
R = 4
L = 2

NC = 2
NS = 16
NW = NC * NS

RN = R * N
NP = 10240

EP = E // NW
P_CH = 2000
P_NCH = EP // P_CH

ES = E // NS
S_CH = 4000
S_NCH = ES // S_CH
GB = 112
NPAD = N + 112
ROWS_PER_TILE = NPAD // NS
OUT_ROWS = 624
OUT_REM = N - NS * OUT_ROWS


def _sc_mesh():
    return plsc.VectorSubcoreMesh(
        core_axis_name="c", subcore_axis_name="s", num_cores=NC,
        num_subcores=NS)


LCAP = 11200
STG = 2240
BPS = STG // GB
NSTG = LCAP // STG


def _deg_body(pk_hbm, out_hbm, lists_hbm, cnt_hbm, deg_v, pv, lx, cv):
    c = lax.axis_index("c")
    s = lax.axis_index("s")
    wid = s * NC + c

    zf = jnp.zeros((16,), jnp.float32)
    ones = jnp.full((16,), 1.0, jnp.float32)

    def zero_step(k, _):
        deg_v[pl.ds(k * 16, 16)] = zf
        return 0

    lax.fori_loop(0, 2 * R * NP // 16, zero_step, 0)

    base0 = wid * EP

    def chunk(k, cnts):
        base = base0 + k * P_CH
        pltpu.sync_copy(pk_hbm.at[pl.ds(base, P_CH)], pv)

        def step(j, cnts):
            w16 = pv[pl.ds(j * 16, 16)]
            e16 = w16 >> 28
            s16 = (w16 >> 14) & 16383
            d16 = w16 & 16383
            idx_out = e16 * NP + s16
            idx_in = (R * NP + e16 * NP) + d16
            plsc.addupdate_scatter(deg_v, [idx_out], ones)
            plsc.addupdate_scatter(deg_v, [idx_in], ones)
            out = []
            for r in range(R):
                m = e16 == r
                plsc.store_compressed(lx.at[pl.ds(r * LCAP + cnts[r], 16)],
                                      w16, mask=m)
                out.append(cnts[r] + plsc.all_reduce_population_count(m)[0])
            return tuple(out)

        return lax.fori_loop(0, P_CH // 16, step, cnts)

    z = jnp.int32(0)
    cnts = lax.fori_loop(0, P_NCH, chunk, (z, z, z, z))

    dummy = jnp.full((16,), N, jnp.int32)
    lanes = lax.iota(jnp.int32, 16)
    cvec = jnp.zeros((16,), jnp.int32)
    for r in range(R):
        for j in range(GB // 16):
            lx[pl.ds(r * LCAP + cnts[r] + j * 16, 16)] = dummy
        cvec = jnp.where(lanes == r, cnts[r], cvec)
    cv[pl.ds(0, 16)] = cvec
    pltpu.sync_copy(cv, cnt_hbm.at[pl.ds(wid * 16, 16)])
    pltpu.sync_copy(lx, lists_hbm.at[pl.ds(wid * R * LCAP, R * LCAP)])
    pltpu.sync_copy(deg_v, out_hbm.at[pl.ds(wid * (2 * R * NP), 2 * R * NP)])


def _deg_call(packed):
    f = functools.partial(
        pl.kernel,
        out_type=[
            jax.ShapeDtypeStruct((NW * 2 * R * NP,), jnp.float32),
            jax.ShapeDtypeStruct((NW * R * LCAP,), jnp.int32),
            jax.ShapeDtypeStruct((NW * 16,), jnp.int32),
        ],
        mesh=_sc_mesh(),
        scratch_types=[
            pltpu.VMEM((2 * R * NP,), jnp.float32),
            pltpu.VMEM((P_CH,), jnp.int32),
            pltpu.VMEM((R * LCAP,), jnp.int32),
            pltpu.VMEM((16,), jnp.int32),
        ],
        compiler_params=pltpu.CompilerParams(needs_layout_passes=False),
    )(_deg_body)
    return f(packed)


NBUF = 3


def _agg_body(hs_hbm, lists_hbm, cnts_hbm, out_hbm,
              pkx, cv, src2, dst2, rows, acc, gsem):
    c = lax.axis_index("c")
    s = lax.axis_index("s")

    zf = jnp.zeros((16,), jnp.float32)

    for p in range(2):
        r = c + NC * p
        rbase = r * N

        def zb_step(i, _):
            for j in range(D // 16):
                rows[0, i, pl.ds(j * 16, 16)] = zf
            return 0

        lax.fori_loop(0, GB, zb_step, 0)
        zoff = s * ROWS_PER_TILE
        for t in range(ROWS_PER_TILE // GB):
            pltpu.sync_copy(rows.at[0],
                            acc.at[pl.ds(zoff + t * GB, GB)])
        zrem = ROWS_PER_TILE - (ROWS_PER_TILE // GB) * GB
        pltpu.sync_copy(
            rows.at[0, pl.ds(0, zrem)],
            acc.at[pl.ds(zoff + (ROWS_PER_TILE // GB) * GB, zrem)])
        plsc.subcore_barrier()

        def wait_gather(buf):
            pltpu.make_async_copy(hs_hbm.at[src2.at[buf]],
                                  rows.at[buf], gsem).wait()

        def seg_loop(seg, fc):
            wid = 2 * s + seg
            pltpu.sync_copy(cnts_hbm.at[pl.ds(wid * 16, 16)], cv)
            lanes = lax.iota(jnp.int32, 16)
            cvv = cv[pl.ds(0, 16)]
            cnt = jnp.sum(jnp.where(lanes == r, cvv, 0))
            nbt = (cnt + (GB - 1)) // GB
            lbase = (wid * R + r) * LCAP

            def stage_loop(h, fc):
                pltpu.sync_copy(
                    lists_hbm.at[pl.ds(lbase + h * STG, STG)], pkx)
                nbh = jnp.clip(nbt - h * BPS, 0, BPS)

                def fire(q, fc):
                    b = lax.rem(fc, NBUF)
                    for j in range(GB // 16):
                        w = pkx[pl.ds(q * GB + j * 16, 16)]
                        src2[b, pl.ds(j * 16, 16)] = \
                            ((w >> 14) & 16383) + rbase
                        dst2[b, pl.ds(j * 16, 16)] = w & 16383
                    pltpu.async_copy(hs_hbm.at[src2.at[b]], rows.at[b], gsem)

                    @pl.when(fc >= 1)
                    def _():
                        bp = lax.rem(fc + (NBUF - 1), NBUF)
                        wait_gather(bp)
                        pltpu.sync_copy(rows.at[bp], acc.at[dst2.at[bp]],
                                        add=True)

                    return fc + 1

                return lax.fori_loop(0, nbh, fire, fc)

            return lax.fori_loop(0, NSTG, stage_loop, fc)

        fc = lax.fori_loop(0, 2, seg_loop, jnp.int32(0))

        @pl.when(fc >= 1)
        def _():
            blast = lax.rem(fc + (NBUF - 1), NBUF)
            wait_gather(blast)
            pltpu.sync_copy(rows.at[blast], acc.at[dst2.at[blast]], add=True)

        plsc.subcore_barrier()
        pltpu.sync_copy(acc.at[pl.ds(s * OUT_ROWS, OUT_ROWS)],
                        out_hbm.at[pl.ds(rbase + s * OUT_ROWS, OUT_ROWS)])

        @pl.when(s == 0)
        def _():
            pltpu.sync_copy(
                acc.at[pl.ds(NS * OUT_ROWS, OUT_REM)],
                out_hbm.at[pl.ds(rbase + NS * OUT_ROWS, OUT_REM)])

        plsc.subcore_barrier()


def _agg_call(hs, lists, cnts):
    f = functools.partial(
        pl.kernel,
        out_type=jax.ShapeDtypeStruct((RN, D), jnp.float32),
        mesh=_sc_mesh(),
        scratch_types=[
            pltpu.VMEM((STG,), jnp.int32),
            pltpu.VMEM((16,), jnp.int32),
            pltpu.VMEM((NBUF, GB), jnp.int32),
            pltpu.VMEM((NBUF, GB), jnp.int32),
            pltpu.VMEM((NBUF, GB, D), jnp.float32),
            pltpu.VMEM_SHARED((NPAD, D), jnp.float32),
            pltpu.SemaphoreType.DMA,
        ],
        compiler_params=pltpu.CompilerParams(needs_layout_passes=False),
    )(_agg_body)
    return f(hs, lists, cnts)


def _norm_body(degparts_ref, norm_ref):
    seg = 2 * R * NP
    deg = degparts_ref[pl.ds(0, seg)]
    for w in range(1, NW):
        deg = deg + degparts_ref[pl.ds(w * seg, seg)]
    norm_ref[...] = lax.rsqrt(jnp.maximum(deg, 1.0))


def _norm_call(degparts):
    return pl.pallas_call(
        _norm_body,
        out_shape=jax.ShapeDtypeStruct((2 * R * NP,), jnp.float32),
    )(degparts)


NB2 = 2000


def _hs_body(feats_ref, ns_ref, hs_ref):
    feats = feats_ref[...]
    ns = ns_ref[...]
    for r in range(R):
        hs_ref[r] = feats * ns[r]


def _hs_call(features, ns_view):
    return pl.pallas_call(
        _hs_body,
        grid=(N // NB2,),
        in_specs=[
            pl.BlockSpec((NB2, D), lambda i: (i, 0)),
            pl.BlockSpec((R, NB2, 1), lambda i: (0, i, 0)),
        ],
        out_specs=pl.BlockSpec((R, NB2, D), lambda i: (0, i, 0)),
        out_shape=jax.ShapeDtypeStruct((R, N, D), jnp.float32),
    )(features, ns_view)


NB = 2000


def _layer_body(is_last, h_ref, agg_ref, nd_ref, ns_ref, w_ref, b_ref,
                fcw_ref, fcb_ref, g_ref, beta_ref, feats_ref, hout_ref,
                *maybe_hs_out):
    agg = agg_ref[...]
    nd = nd_ref[...]
    a = agg * nd
    w = w_ref[...]
    conv = jnp.zeros((NB, D), jnp.float32)
    for r in range(R):
        conv = conv + jnp.dot(a[r], w[r],
                              precision=lax.Precision.DEFAULT)
    conv = conv + jnp.sum(b_ref[...], axis=0)[None, :]
    logits = jnp.sum(conv * fcw_ref[...], axis=1, keepdims=True) + fcb_ref[0]
    attn = jax.nn.sigmoid(logits)
    h2 = h_ref[...] + attn * conv
    mu = jnp.mean(h2, axis=-1, keepdims=True)
    var = jnp.mean((h2 - mu) ** 2, axis=-1, keepdims=True)
    y = (h2 - mu) * lax.rsqrt(var + 1e-5) * g_ref[...][None, :] \
        + beta_ref[...][None, :]
    h3 = jnp.maximum(y, 0.0)
    if is_last:
        h3 = h3 + feats_ref[...]
        hout_ref[...] = h3
    else:
        hout_ref[...] = h3
        maybe_hs_out[0][...] = h3[None, :, :] * ns_ref[...]


def _layer_call(h, agg, nd, ns, w, b, fcw, fcb, g, beta, feats, is_last):
    grid = (N // NB,)
    in_specs = [
        pl.BlockSpec((NB, D), lambda i: (i, 0)),
        pl.BlockSpec((R, NB, D), lambda i: (0, i, 0)),
        pl.BlockSpec((R, NB, 1), lambda i: (0, i, 0)),
        pl.BlockSpec((R, NB, 1), lambda i: (0, i, 0)),
        pl.BlockSpec((R, D, D), lambda i: (0, 0, 0)),
        pl.BlockSpec((R, D), lambda i: (0, 0)),
        pl.BlockSpec((1, D), lambda i: (0, 0)),
        pl.BlockSpec((1,), lambda i: (0,)),
        pl.BlockSpec((D,), lambda i: (0,)),
        pl.BlockSpec((D,), lambda i: (0,)),
        pl.BlockSpec((NB, D), lambda i: (i, 0)),
    ]
    out_shape = [jax.ShapeDtypeStruct((N, D), jnp.float32)]
    out_specs = [pl.BlockSpec((NB, D), lambda i: (i, 0))]
    if not is_last:
        out_shape.append(jax.ShapeDtypeStruct((R, N, D), jnp.float32))
        out_specs.append(pl.BlockSpec((R, NB, D), lambda i: (0, i, 0)))
    return pl.pallas_call(
        functools.partial(_layer_body, is_last),
        grid=grid,
        in_specs=in_specs,
        out_specs=out_specs,
        out_shape=out_shape,
    )(h, agg, nd, ns, w, b, fcw, fcb, g, beta, feats)


def kernel(features, edge_index, edge_type, W, B, fc_w, fc_b, ln_gamma,
           ln_beta):
    src = edge_index[0]
    dst = edge_index[1]
    packed = (edge_type.astype(jnp.int32) << 28) | (src << 14) | dst

    degflat, lists, cnts = _deg_call(packed)
    norm_flat = _norm_call(degflat)
    ns = norm_flat[:R * NP].reshape(R, NP, 1)
    nd = norm_flat[R * NP:].reshape(R, NP, 1)
    hs = _hs_call(features, ns).reshape(RN, D)

    fcw_t = fc_w.reshape(1, D)
    h = features
    for l in range(L):
        agg3 = _agg_call(hs, lists, cnts).reshape(R, N, D)
        is_last = l == L - 1
        outs = _layer_call(h, agg3, nd, ns, W[l], B[l], fcw_t, fc_b,
                           ln_gamma[l], ln_beta[l], features, is_last)
        if is_last:
            h = outs[0]
        else:
            h, hs4 = outs
            hs = hs4.reshape(RN, D)
    return h
